# baseline (speedup 1.0000x reference)
import jax
import jax.numpy as jnp
from jax.experimental import pallas as pl
from jax.experimental.pallas import tpu as pltpu

_ALPHA = 0.2
_HIGH = jax.lax.Precision.DEFAULT


def _support_kernel(x_ref, w_ref, out_ref):
    out_ref[...] = jnp.dot(x_ref[...], w_ref[...],
                           preferred_element_type=jnp.float32,
                           precision=_HIGH)


def _main_kernel(adj_ref, sup_ref, b_ref, mu_ref, z_ref, q_ref):
    z = jnp.dot(adj_ref[...], sup_ref[...],
                preferred_element_type=jnp.float32,
                precision=_HIGH) + b_ref[...]
    z_ref[...] = z
    mu = mu_ref[...]
    zsq = jnp.sum(z * z, axis=1, keepdims=True)
    musq = jnp.sum(mu * mu, axis=1)
    cross = jax.lax.dot_general(
        z, mu, dimension_numbers=(((1,), (1,)), ((), ())),
        preferred_element_type=jnp.float32, precision=_HIGH)
    d2 = zsq + musq[None, :] - 2.0 * cross
    q = 1.0 / (1.0 + d2 / _ALPHA + 1e-8)
    q = q ** (_ALPHA + 1.0)
    q_ref[...] = q / jnp.sum(q, axis=1, keepdims=True)


def kernel(x, adj, W, b, mu):
    n, nfeat = x.shape
    nhid = W.shape[1]
    nclust = mu.shape[0]

    bms = 2000
    support = pl.pallas_call(
        _support_kernel,
        grid=(n // bms,),
        in_specs=[
            pl.BlockSpec((bms, nfeat), lambda i: (i, 0)),
            pl.BlockSpec((nfeat, nhid), lambda i: (0, 0)),
        ],
        out_specs=pl.BlockSpec((bms, nhid), lambda i: (i, 0)),
        out_shape=jax.ShapeDtypeStruct((n, nhid), jnp.float32),
    )(x, W)

    bm = 200
    z, q = pl.pallas_call(
        _main_kernel,
        grid=(n // bm,),
        in_specs=[
            pl.BlockSpec((bm, n), lambda i: (i, 0)),
            pl.BlockSpec((n, nhid), lambda i: (0, 0)),
            pl.BlockSpec((1, nhid), lambda i: (0, 0)),
            pl.BlockSpec((nclust, nhid), lambda i: (0, 0)),
        ],
        out_specs=[
            pl.BlockSpec((bm, nhid), lambda i: (i, 0)),
            pl.BlockSpec((bm, nclust), lambda i: (i, 0)),
        ],
        out_shape=[
            jax.ShapeDtypeStruct((n, nhid), jnp.float32),
            jax.ShapeDtypeStruct((n, nclust), jnp.float32),
        ],
        compiler_params=pltpu.CompilerParams(
            dimension_semantics=("parallel",)),
    )(adj, support, b.reshape(1, nhid), mu)
    return z, q


# speedup vs baseline: 1.0637x; 1.0637x over previous
import jax
import jax.numpy as jnp
from jax.experimental import pallas as pl
from jax.experimental.pallas import tpu as pltpu

_ALPHA = 0.2
_HIGH = jax.lax.Precision.DEFAULT


def _support_kernel(x_ref, w_ref, out_ref):
    out_ref[...] = jnp.dot(x_ref[...], w_ref[...],
                           preferred_element_type=jnp.float32,
                           precision=_HIGH)


def _main_kernel(adj_ref, sup_ref, b_ref, mu_ref, z_ref, q_ref):
    z = jnp.dot(adj_ref[...], sup_ref[...],
                preferred_element_type=jnp.float32,
                precision=_HIGH) + b_ref[...]
    z_ref[...] = z
    mu = mu_ref[...]
    zsq = jnp.sum(z * z, axis=1, keepdims=True)
    musq = jnp.sum(mu * mu, axis=1)
    cross = jax.lax.dot_general(
        z, mu, dimension_numbers=(((1,), (1,)), ((), ())),
        preferred_element_type=jnp.float32, precision=_HIGH)
    d2 = zsq + musq[None, :] - 2.0 * cross
    q = 1.0 / (1.0 + d2 / _ALPHA + 1e-8)
    q = q ** (_ALPHA + 1.0)
    q_ref[...] = q / jnp.sum(q, axis=1, keepdims=True)


def kernel(x, adj, W, b, mu):
    n, nfeat = x.shape
    nhid = W.shape[1]
    nclust = mu.shape[0]

    bms = 2000
    support = pl.pallas_call(
        _support_kernel,
        grid=(n // bms,),
        in_specs=[
            pl.BlockSpec((bms, nfeat), lambda i: (i, 0)),
            pl.BlockSpec((nfeat, nhid), lambda i: (0, 0)),
        ],
        out_specs=pl.BlockSpec((bms, nhid), lambda i: (i, 0)),
        out_shape=jax.ShapeDtypeStruct((n, nhid), jnp.float32),
    )(x, W)

    bm = 400
    z, q = pl.pallas_call(
        _main_kernel,
        grid=(n // bm,),
        in_specs=[
            pl.BlockSpec((bm, n), lambda i: (i, 0)),
            pl.BlockSpec((n, nhid), lambda i: (0, 0)),
            pl.BlockSpec((1, nhid), lambda i: (0, 0)),
            pl.BlockSpec((nclust, nhid), lambda i: (0, 0)),
        ],
        out_specs=[
            pl.BlockSpec((bm, nhid), lambda i: (i, 0)),
            pl.BlockSpec((bm, nclust), lambda i: (i, 0)),
        ],
        out_shape=[
            jax.ShapeDtypeStruct((n, nhid), jnp.float32),
            jax.ShapeDtypeStruct((n, nclust), jnp.float32),
        ],
        compiler_params=pltpu.CompilerParams(
            dimension_semantics=("parallel",)),
    )(adj, support, b.reshape(1, nhid), mu)
    return z, q


# speedup vs baseline: 1.0921x; 1.0266x over previous
import jax
import jax.numpy as jnp
from jax.experimental import pallas as pl
from jax.experimental.pallas import tpu as pltpu

_ALPHA = 0.2
_PREC = jax.lax.Precision.DEFAULT


def _main_kernel(adj_ref, x_ref, w_ref, b_ref, mu_ref, z_ref, q_ref,
                 sup_ref):
    @pl.when(pl.program_id(0) == 0)
    def _():
        sup_ref[...] = jnp.dot(x_ref[...], w_ref[...],
                               preferred_element_type=jnp.float32,
                               precision=_PREC)

    z = jnp.dot(adj_ref[...], sup_ref[...],
                preferred_element_type=jnp.float32,
                precision=_PREC) + b_ref[...]
    z_ref[...] = z
    mu = mu_ref[...]
    zsq = jnp.sum(z * z, axis=1, keepdims=True)
    musq = jnp.sum(mu * mu, axis=1)
    cross = jax.lax.dot_general(
        z, mu, dimension_numbers=(((1,), (1,)), ((), ())),
        preferred_element_type=jnp.float32, precision=_PREC)
    d2 = zsq + musq[None, :] - 2.0 * cross
    q = 1.0 / (1.0 + d2 / _ALPHA + 1e-8)
    q = q ** (_ALPHA + 1.0)
    q_ref[...] = q / jnp.sum(q, axis=1, keepdims=True)


def kernel(x, adj, W, b, mu):
    n, nfeat = x.shape
    nhid = W.shape[1]
    nclust = mu.shape[0]

    bm = 400
    z, q = pl.pallas_call(
        _main_kernel,
        grid=(n // bm,),
        in_specs=[
            pl.BlockSpec((bm, n), lambda i: (i, 0)),
            pl.BlockSpec((n, nfeat), lambda i: (0, 0)),
            pl.BlockSpec((nfeat, nhid), lambda i: (0, 0)),
            pl.BlockSpec((1, nhid), lambda i: (0, 0)),
            pl.BlockSpec((nclust, nhid), lambda i: (0, 0)),
        ],
        out_specs=[
            pl.BlockSpec((bm, nhid), lambda i: (i, 0)),
            pl.BlockSpec((bm, nclust), lambda i: (i, 0)),
        ],
        out_shape=[
            jax.ShapeDtypeStruct((n, nhid), jnp.float32),
            jax.ShapeDtypeStruct((n, nclust), jnp.float32),
        ],
        scratch_shapes=[pltpu.VMEM((n, nhid), jnp.float32)],
        compiler_params=pltpu.CompilerParams(
            dimension_semantics=("arbitrary",)),
    )(adj, x, W, b.reshape(1, nhid), mu)
    return z, q
